# baseline (speedup 1.0000x reference)
import functools
import math

import jax
import jax.numpy as jnp
from jax import lax
from jax.experimental import pallas as pl
from jax.experimental.pallas import tpu as pltpu
from jax.experimental.pallas import tpu_sc as plsc

_MARGIN = 0.5
_COS_M = math.cos(_MARGIN)
_SIN_M = math.sin(_MARGIN)
_B = 1024
_C = 100000

_NC = 2
_NS = 16
_NW = _NC * _NS
_BPW = _B // _NW
_L = 16
_G = 128

_RB = 32
_NSTEPS = _B // _RB


def _sc_gather(cosg, labels):
    mesh = plsc.VectorSubcoreMesh(core_axis_name="c", subcore_axis_name="s")

    @functools.partial(
        pl.kernel,
        mesh=mesh,
        out_type=jax.ShapeDtypeStruct((_B, _G), jnp.float32),
        scratch_types=[
            pltpu.VMEM((_BPW,), jnp.int32),
            pltpu.VMEM((_BPW,), jnp.int32),
            pltpu.VMEM((_BPW, _G), jnp.float32),
            pltpu.SemaphoreType.DMA,
        ],
    )
    def k(cosg_hbm, lab_hbm, out_hbm, lab_v, idx_v, rows_v, sem):
        wid = lax.axis_index("s") * _NC + lax.axis_index("c")
        base = wid * _BPW
        pltpu.sync_copy(lab_hbm.at[pl.ds(base, _BPW)], lab_v)
        for ch in range(_BPW // _L):
            lab = lab_v[pl.ds(ch * _L, _L)]
            iot = lax.broadcasted_iota(jnp.int32, (_L,), 0)
            flat = (base + ch * _L + iot) * _C + lab
            idx_v[pl.ds(ch * _L, _L)] = lax.shift_right_logical(flat, 7)
        pltpu.async_copy(cosg_hbm.at[idx_v], rows_v, sem).wait()
        pltpu.sync_copy(rows_v, out_hbm.at[pl.ds(base, _BPW)])

    return k(cosg, labels)


def _tc_body(x_ref, g_ref, lab_ref, out_ref, acc_ref):
    j = pl.program_id(0)

    @pl.when(j == 0)
    def _init():
        acc_ref[0] = 0.0

    x = x_ref[...]
    row_sum = jnp.sum(x, axis=1)
    lab = lab_ref[...][:, 0]
    rows = j * _RB + lax.broadcasted_iota(jnp.int32, (_RB,), 0)
    lane = lax.bitwise_and(rows * _C + lab, _G - 1)
    sel = lax.broadcasted_iota(jnp.int32, (_RB, _G), 1) == lane[:, None]
    c = jnp.sum(jnp.where(sel, g_ref[...], 0.0), axis=1)
    sine = jnp.sqrt(jnp.maximum(1.0 - c * c, 0.0))
    m = c * _COS_M - sine * _SIN_M
    adj = row_sum - jnp.exp(c) + jnp.exp(m)
    nll = jnp.log(adj) - m
    acc_ref[0] = acc_ref[0] + jnp.sum(nll)

    @pl.when(j == _NSTEPS - 1)
    def _fin():
        out_ref[0, 0] = acc_ref[0] * (1.0 / _B)


def _tc_loss(cosine, grp, labels):
    return pl.pallas_call(
        _tc_body,
        grid=(_NSTEPS,),
        in_specs=[
            pl.BlockSpec((_RB, _C), lambda j: (j, 0)),
            pl.BlockSpec((_RB, _G), lambda j: (j, 0)),
            pl.BlockSpec((_RB, 1), lambda j: (j, 0)),
        ],
        out_specs=pl.BlockSpec(memory_space=pltpu.SMEM),
        out_shape=jax.ShapeDtypeStruct((1, 1), jnp.float32),
        scratch_shapes=[pltpu.SMEM((1,), jnp.float32)],
    )(cosine, grp, labels.reshape(_B, 1))


def kernel(cosine, labels):
    labels = labels.astype(jnp.int32)
    grp = cosine[:, :_G]
    loss = _tc_loss(cosine, grp, labels)
    return loss[0, 0]


# speedup vs baseline: 3.2149x; 3.2149x over previous
import functools
import math

import jax
import jax.numpy as jnp
from jax import lax
from jax.experimental import pallas as pl
from jax.experimental.pallas import tpu as pltpu
from jax.experimental.pallas import tpu_sc as plsc

_MARGIN = 0.5
_COS_M = math.cos(_MARGIN)
_SIN_M = math.sin(_MARGIN)
_B = 1024
_C = 100000

_NC = 2
_NS = 16
_NW = _NC * _NS
_BPW = _B // _NW

_CB = 2000
_NSTEPS = _C // _CB


def _sc_gather(xt, labels):
    mesh = plsc.VectorSubcoreMesh(core_axis_name="c", subcore_axis_name="s")

    @functools.partial(
        pl.kernel,
        mesh=mesh,
        out_type=jax.ShapeDtypeStruct((_B, _B), jnp.float32),
        scratch_types=[
            pltpu.VMEM((_BPW,), jnp.int32),
            pltpu.VMEM((_BPW, _B), jnp.float32),
            pltpu.SemaphoreType.DMA,
        ],
    )
    def k(xt_hbm, lab_hbm, out_hbm, lab_v, rows_v, sem):
        wid = lax.axis_index("s") * _NC + lax.axis_index("c")
        base = wid * _BPW
        pltpu.sync_copy(lab_hbm.at[pl.ds(base, _BPW)], lab_v)
        pltpu.async_copy(xt_hbm.at[lab_v], rows_v, sem).wait()
        pltpu.sync_copy(rows_v, out_hbm.at[pl.ds(base, _BPW)])

    return k(xt, labels)


def _tc_body(x_ref, g_ref, out_ref, acc_ref):
    j = pl.program_id(0)

    @pl.when(j == 0)
    def _init():
        acc_ref[...] = jnp.zeros_like(acc_ref)

    ex = jnp.exp(x_ref[...])
    acc_ref[...] = acc_ref[...] + jnp.sum(ex, axis=0)

    @pl.when(j == _NSTEPS - 1)
    def _fin():
        row_sum = acc_ref[...]
        eye = (lax.broadcasted_iota(jnp.int32, (_B, _B), 0)
               == lax.broadcasted_iota(jnp.int32, (_B, _B), 1))
        c = jnp.sum(jnp.where(eye, g_ref[...], 0.0), axis=1)
        sine = jnp.sqrt(jnp.maximum(1.0 - c * c, 0.0))
        m = c * _COS_M - sine * _SIN_M
        adj = row_sum - jnp.exp(c) + jnp.exp(m)
        nll = jnp.log(adj) - m
        out_ref[0, 0] = jnp.sum(nll) * (1.0 / _B)


def _tc_loss(xt, grp):
    return pl.pallas_call(
        _tc_body,
        grid=(_NSTEPS,),
        in_specs=[
            pl.BlockSpec((_CB, _B), lambda j: (j, 0)),
            pl.BlockSpec((_B, _B), lambda j: (0, 0)),
        ],
        out_specs=pl.BlockSpec(memory_space=pltpu.SMEM),
        out_shape=jax.ShapeDtypeStruct((1, 1), jnp.float32),
        scratch_shapes=[pltpu.VMEM((_B,), jnp.float32)],
    )(xt, grp)


def kernel(cosine, labels):
    labels = labels.astype(jnp.int32)
    xt = cosine.T
    grp = _sc_gather(xt, labels)
    loss = _tc_loss(xt, grp)
    return loss[0, 0]


# speedup vs baseline: 3.2305x; 1.0049x over previous
import functools
import math

import jax
import jax.numpy as jnp
from jax import lax
from jax.experimental import pallas as pl
from jax.experimental.pallas import tpu as pltpu
from jax.experimental.pallas import tpu_sc as plsc

_MARGIN = 0.5
_COS_M = math.cos(_MARGIN)
_SIN_M = math.sin(_MARGIN)
_B = 1024
_C = 100000

_NC = 2
_NS = 16
_NW = _NC * _NS
_BPW = _B // _NW

_CB = 4000
_NSTEPS = _C // _CB


def _sc_gather(xt, labels):
    mesh = plsc.VectorSubcoreMesh(core_axis_name="c", subcore_axis_name="s")

    @functools.partial(
        pl.kernel,
        mesh=mesh,
        out_type=jax.ShapeDtypeStruct((_B, _B), jnp.float32),
        scratch_types=[
            pltpu.VMEM((_BPW,), jnp.int32),
            pltpu.VMEM((_BPW, _B), jnp.float32),
            pltpu.SemaphoreType.DMA,
        ],
    )
    def k(xt_hbm, lab_hbm, out_hbm, lab_v, rows_v, sem):
        wid = lax.axis_index("s") * _NC + lax.axis_index("c")
        base = wid * _BPW
        pltpu.sync_copy(lab_hbm.at[pl.ds(base, _BPW)], lab_v)
        pltpu.async_copy(xt_hbm.at[lab_v], rows_v, sem).wait()
        pltpu.sync_copy(rows_v, out_hbm.at[pl.ds(base, _BPW)])

    return k(xt, labels)


def _tc_body(x_ref, g_ref, out_ref, acc_ref):
    j = pl.program_id(0)

    @pl.when(j == 0)
    def _init():
        acc_ref[...] = jnp.zeros_like(acc_ref)

    ex = jnp.exp(x_ref[...])
    acc_ref[...] = acc_ref[...] + jnp.sum(ex, axis=0)

    @pl.when(j == _NSTEPS - 1)
    def _fin():
        row_sum = acc_ref[...]
        eye = (lax.broadcasted_iota(jnp.int32, (_B, _B), 0)
               == lax.broadcasted_iota(jnp.int32, (_B, _B), 1))
        c = jnp.sum(jnp.where(eye, g_ref[...], 0.0), axis=1)
        sine = jnp.sqrt(jnp.maximum(1.0 - c * c, 0.0))
        m = c * _COS_M - sine * _SIN_M
        adj = row_sum - jnp.exp(c) + jnp.exp(m)
        nll = jnp.log(adj) - m
        out_ref[0, 0] = jnp.sum(nll) * (1.0 / _B)


def _tc_loss(xt, grp):
    return pl.pallas_call(
        _tc_body,
        grid=(_NSTEPS,),
        in_specs=[
            pl.BlockSpec((_CB, _B), lambda j: (j, 0)),
            pl.BlockSpec((_B, _B), lambda j: (0, 0)),
        ],
        out_specs=pl.BlockSpec(memory_space=pltpu.SMEM),
        out_shape=jax.ShapeDtypeStruct((1, 1), jnp.float32),
        scratch_shapes=[pltpu.VMEM((_B,), jnp.float32)],
    )(xt, grp)


def kernel(cosine, labels):
    labels = labels.astype(jnp.int32)
    xt = cosine.T
    grp = _sc_gather(xt, labels)
    loss = _tc_loss(xt, grp)
    return loss[0, 0]
